# baseline (speedup 1.0000x reference)
import functools

import jax
import jax.numpy as jnp
from jax import lax
from jax.experimental import pallas as pl
from jax.experimental.pallas import tpu as pltpu
from jax.experimental.pallas import tpu_sc as plsc

B = 1024
N0 = 25
N1 = 10
D = 128
NG = D // 16


def _sc_gather_sums(table, idxq, idx0t, idx1t):
    info = plsc.get_sparse_core_info()
    nc, ns = info.num_cores, info.num_subcores
    nw = nc * ns
    bw = B // nw
    mesh = plsc.VectorSubcoreMesh(core_axis_name="c", subcore_axis_name="s")

    @functools.partial(
        pl.kernel,
        mesh=mesh,
        out_type=[
            jax.ShapeDtypeStruct((B, D), jnp.float32),
            jax.ShapeDtypeStruct((B, D), jnp.float32),
            jax.ShapeDtypeStruct((B, D), jnp.float32),
        ],
        scratch_types=[
            pltpu.VMEM((bw,), jnp.int32),
            pltpu.VMEM((N0, 4 * bw), jnp.int32),
            pltpu.VMEM((N1, N0, 4 * bw), jnp.int32),
            pltpu.VMEM((5, bw, D), jnp.float32),
            pltpu.VMEM((bw, D), jnp.float32),
            pltpu.VMEM((bw, D), jnp.float32),
            pltpu.VMEM((bw, D), jnp.float32),
            pltpu.SemaphoreType.DMA,
            pltpu.SemaphoreType.DMA,
            pltpu.SemaphoreType.DMA,
            pltpu.SemaphoreType.DMA,
            pltpu.SemaphoreType.DMA,
            pltpu.SemaphoreType.DMA,
        ],
    )
    def k(table_h, idxq_h, idx0_h, idx1_h, ev_h, su_h, sn_h,
          idxq_v, idx0_v, idx1_v, buf, ev_v, su_v, sn_v,
          sg0, sg1, sg2, sg3, sg4, sev):
        sg = (sg0, sg1, sg2, sg3, sg4)
        wid = lax.axis_index("s") * nc + lax.axis_index("c")
        base = wid * bw
        cb = (wid // 4) * (4 * bw)
        off = (wid % 4) * bw
        pltpu.sync_copy(idxq_h.at[pl.ds(base, bw)], idxq_v)
        pltpu.sync_copy(idx0_h.at[:, pl.ds(cb, 4 * bw)], idx0_v)
        pltpu.sync_copy(idx1_h.at[:, :, pl.ds(cb, 4 * bw)], idx1_v)
        evcp = pltpu.async_copy(table_h.at[idxq_v], ev_v, sev)

        zeros = jnp.zeros((16,), jnp.float32)
        for r in range(bw):
            for g in range(NG):
                su_v[r, pl.ds(g * 16, 16)] = zeros
                sn_v[r, pl.ds(g * 16, 16)] = zeros

        def slicer1(p):
            return idx1_v.at[p // N0, p % N0, pl.ds(off, bw)]

        def slicer0(p):
            return idx0_v.at[p, pl.ds(off, bw)]

        def fire(slicer, p, s):
            pltpu.async_copy(table_h.at[slicer(p)], buf.at[s], sg[s])

        def wait(slicer, p, s):
            pltpu.make_async_copy(table_h.at[slicer(p)], buf.at[s],
                                  sg[s]).wait()

        def accum(s, acc_v, unroll):
            def body(r, c):
                for g in range(NG):
                    plsc.addupdate(acc_v.at[r, pl.ds(g * 16, 16)],
                                   buf[s, r, pl.ds(g * 16, 16)])
                return c
            lax.fori_loop(0, bw, body, 0, unroll=unroll)

        def ring(slicer, npairs, acc_v):
            for s in range(4):
                fire(slicer, s, s)

            def grp(it, carry):
                pb = 5 * it
                for s in range(5):
                    p = pb + s

                    @pl.when(p + 4 < npairs)
                    def _():
                        fire(slicer, p + 4, (s + 4) % 5)

                    wait(slicer, p, s)
                    accum(s, acc_v, 4)
                return carry

            lax.fori_loop(0, npairs // 5, grp, 0)

        ring(slicer1, N1 * N0, sn_v)
        ring(slicer0, N0, su_v)

        evcp.wait()
        pltpu.sync_copy(ev_v, ev_h.at[pl.ds(base, bw)])
        pltpu.sync_copy(su_v, su_h.at[pl.ds(base, bw)])
        pltpu.sync_copy(sn_v, sn_h.at[pl.ds(base, bw)])

    return k(table, idxq, idx0t, idx1t)


def _tc_body(ev_ref, su_ref, sn_ref, w1_ref, w0_ref, b0_ref, out_ref):
    su = su_ref[...] * (1.0 / N0)
    sn = sn_ref[...] * (1.0 / (N0 * N1))
    agg = (jnp.dot(su, w1_ref[0:D, :], preferred_element_type=jnp.float32)
           + jnp.dot(sn, w1_ref[D:2 * D, :], preferred_element_type=jnp.float32))
    z = (jnp.dot(ev_ref[...], w0_ref[0:D, :], preferred_element_type=jnp.float32)
         + jnp.dot(agg, w0_ref[D:2 * D, :], preferred_element_type=jnp.float32)
         + b0_ref[...])
    out_ref[...] = jax.nn.sigmoid(z)


def _tc_combine(ev, su, sn, W1, W0, b0):
    return pl.pallas_call(
        _tc_body,
        out_shape=jax.ShapeDtypeStruct((B, D), jnp.float32),
    )(ev, su, sn, W1, W0, b0)


def kernel(inputs, nbr0, nbr1, embed_table, W0, b0, W1):
    idx0t = jnp.transpose(nbr0, (1, 0))
    idx1t = jnp.transpose(nbr1, (2, 1, 0))
    ev, su, sn = _sc_gather_sums(embed_table, inputs, idx0t, idx1t)
    return _tc_combine(ev, su, sn, W1, W0, b0.reshape(1, D))


# speedup vs baseline: 1.2904x; 1.2904x over previous
import functools

import jax
import jax.numpy as jnp
from jax import lax
from jax.experimental import pallas as pl
from jax.experimental.pallas import tpu as pltpu
from jax.experimental.pallas import tpu_sc as plsc

B = 1024
N0 = 25
N1 = 10
D = 128
NG = D // 16


def _sc_gather_sums(table, idxq, idx0t, idx1t):
    info = plsc.get_sparse_core_info()
    nc, ns = info.num_cores, info.num_subcores
    nw = nc * ns
    bw = B // nw
    mesh = plsc.VectorSubcoreMesh(core_axis_name="c", subcore_axis_name="s")

    @functools.partial(
        pl.kernel,
        mesh=mesh,
        out_type=[
            jax.ShapeDtypeStruct((B, D), jnp.float32),
            jax.ShapeDtypeStruct((B, D), jnp.float32),
            jax.ShapeDtypeStruct((B, D), jnp.float32),
        ],
        scratch_types=[
            pltpu.VMEM((bw,), jnp.int32),
            pltpu.VMEM((N0, 4 * bw), jnp.int32),
            pltpu.VMEM((N1, N0, 4 * bw), jnp.int32),
            pltpu.VMEM((3, 5, bw, D), jnp.float32),
            pltpu.VMEM((bw, D), jnp.float32),
            pltpu.VMEM((bw, D), jnp.float32),
            pltpu.VMEM((bw, D), jnp.float32),
            pltpu.SemaphoreType.DMA,
            pltpu.SemaphoreType.DMA,
            pltpu.SemaphoreType.DMA,
            pltpu.SemaphoreType.DMA,
        ],
    )
    def k(table_h, idxq_h, idx0_h, idx1_h, ev_h, su_h, sn_h,
          idxq_v, idx0_v, idx1_v, buf, ev_v, su_v, sn_v,
          sg0, sg1, sg2, sev):
        sg = (sg0, sg1, sg2)
        wid = lax.axis_index("s") * nc + lax.axis_index("c")
        base = wid * bw
        cb = (wid // 4) * (4 * bw)
        off = (wid % 4) * bw
        pltpu.sync_copy(idxq_h.at[pl.ds(base, bw)], idxq_v)
        pltpu.sync_copy(idx0_h.at[:, pl.ds(cb, 4 * bw)], idx0_v)
        pltpu.sync_copy(idx1_h.at[:, :, pl.ds(cb, 4 * bw)], idx1_v)
        evcp = pltpu.async_copy(table_h.at[idxq_v], ev_v, sev)

        zeros = jnp.zeros((16,), jnp.float32)
        for r in range(bw):
            for g in range(NG):
                su_v[r, pl.ds(g * 16, 16)] = zeros
                sn_v[r, pl.ds(g * 16, 16)] = zeros

        def slicer1(p):
            return idx1_v.at[p // N0, p % N0, pl.ds(off, bw)]

        def slicer0(p):
            return idx0_v.at[p, pl.ds(off, bw)]

        def fire(slicer, grp, s):
            for q in range(5):
                pltpu.async_copy(table_h.at[slicer(5 * grp + q)],
                                 buf.at[s, q], sg[s])

        def wait(slicer, grp, s):
            for q in range(5):
                pltpu.make_async_copy(table_h.at[slicer(5 * grp + q)],
                                      buf.at[s, q], sg[s]).wait()

        def accum(s, acc_v):
            def body(r, c):
                for g in range(NG):
                    v = buf[s, 0, r, pl.ds(g * 16, 16)]
                    for q in range(1, 5):
                        v = v + buf[s, q, r, pl.ds(g * 16, 16)]
                    plsc.addupdate(acc_v.at[r, pl.ds(g * 16, 16)], v)
                return c
            lax.fori_loop(0, bw, body, 0, unroll=2)

        def ring(slicer, ngroups, acc_v):
            fire(slicer, 0, 0)
            fire(slicer, 1, 1)
            nfull = (ngroups // 3) * 3

            def grp3(it, carry):
                gb = 3 * it
                for s in range(3):
                    g = gb + s

                    @pl.when(g + 2 < ngroups)
                    def _():
                        fire(slicer, g + 2, (s + 2) % 3)

                    wait(slicer, g, s)
                    accum(s, acc_v)
                return carry

            lax.fori_loop(0, ngroups // 3, grp3, 0)
            for g in range(nfull, ngroups):
                s = g % 3

                @pl.when(g + 2 < ngroups)
                def _():
                    fire(slicer, g + 2, (s + 2) % 3)

                wait(slicer, g, s)
                accum(s, acc_v)

        ring(slicer1, (N1 * N0) // 5, sn_v)
        ring(slicer0, N0 // 5, su_v)

        evcp.wait()
        pltpu.sync_copy(ev_v, ev_h.at[pl.ds(base, bw)])
        pltpu.sync_copy(su_v, su_h.at[pl.ds(base, bw)])
        pltpu.sync_copy(sn_v, sn_h.at[pl.ds(base, bw)])

    return k(table, idxq, idx0t, idx1t)


def _tc_body(ev_ref, su_ref, sn_ref, w1_ref, w0_ref, b0_ref, out_ref):
    su = su_ref[...] * (1.0 / N0)
    sn = sn_ref[...] * (1.0 / (N0 * N1))
    agg = (jnp.dot(su, w1_ref[0:D, :], preferred_element_type=jnp.float32)
           + jnp.dot(sn, w1_ref[D:2 * D, :], preferred_element_type=jnp.float32))
    z = (jnp.dot(ev_ref[...], w0_ref[0:D, :], preferred_element_type=jnp.float32)
         + jnp.dot(agg, w0_ref[D:2 * D, :], preferred_element_type=jnp.float32)
         + b0_ref[...])
    out_ref[...] = jax.nn.sigmoid(z)


def _tc_combine(ev, su, sn, W1, W0, b0):
    return pl.pallas_call(
        _tc_body,
        out_shape=jax.ShapeDtypeStruct((B, D), jnp.float32),
    )(ev, su, sn, W1, W0, b0)


def kernel(inputs, nbr0, nbr1, embed_table, W0, b0, W1):
    idx0t = jnp.transpose(nbr0, (1, 0))
    idx1t = jnp.transpose(nbr1, (2, 1, 0))
    ev, su, sn = _sc_gather_sums(embed_table, inputs, idx0t, idx1t)
    return _tc_combine(ev, su, sn, W1, W0, b0.reshape(1, D))
